# baseline (speedup 1.0000x reference)
import functools

import jax
import jax.numpy as jnp
from jax import lax
from jax.experimental import pallas as pl
from jax.experimental.pallas import tpu as pltpu
from jax.experimental.pallas import tpu_sc as plsc

_M = 100000
_B = 16384
_D = 64
_NT = 32
_HALF = _M // 2
_CH = 2048
_NV = _CH // 16
_NZ = _HALF // 16

_PRE_BR = 2048
_POST_BR = 4096


def _pre_body(idx_ref, src_ref, idxT_ref, logT_ref, negT_ref):
    s = src_ref[...]
    idxT_ref[...] = idx_ref[...].T
    logT_ref[...] = jnp.log(jnp.abs(s)).T
    negT_ref[...] = (s < 0).astype(jnp.int32).T


_pre = pl.pallas_call(
    _pre_body,
    grid=(_B // _PRE_BR,),
    in_specs=[
        pl.BlockSpec((_PRE_BR, _D), lambda i: (i, 0)),
        pl.BlockSpec((_PRE_BR, _D), lambda i: (i, 0)),
    ],
    out_specs=[
        pl.BlockSpec((_D, _PRE_BR), lambda i: (0, i)),
        pl.BlockSpec((_D, _PRE_BR), lambda i: (0, i)),
        pl.BlockSpec((_D, _PRE_BR), lambda i: (0, i)),
    ],
    out_shape=[
        jax.ShapeDtypeStruct((_D, _B), jnp.int32),
        jax.ShapeDtypeStruct((_D, _B), jnp.float32),
        jax.ShapeDtypeStruct((_D, _B), jnp.int32),
    ],
)


_mesh = plsc.VectorSubcoreMesh(core_axis_name="c", subcore_axis_name="s")


@functools.partial(
    pl.kernel,
    mesh=_mesh,
    compiler_params=pltpu.CompilerParams(needs_layout_passes=False),
    out_type=[
        jax.ShapeDtypeStruct((_D * _M,), jnp.float32),
        jax.ShapeDtypeStruct((_D * _M,), jnp.int32),
    ],
    scratch_types=[
        pltpu.VMEM((_HALF,), jnp.float32),
        pltpu.VMEM((_HALF,), jnp.int32),
        pltpu.VMEM((_CH,), jnp.int32),
        pltpu.VMEM((_CH,), jnp.float32),
        pltpu.VMEM((_CH,), jnp.int32),
    ],
)
def _sc_scatter(idxT, logT, negT, laT_out, ncT_out, la, nc, idxbuf, logbuf, negbuf):
    wid = lax.axis_index("s") * 2 + lax.axis_index("c")
    for c2 in range(2):
        j = wid + _NT * c2
        jb = j * _B
        jm = j * _M
        for p in range(2):
            base = p * _HALF

            def zbody(v, carry):
                sl = pl.ds(v * 16, 16)
                la[sl] = jnp.zeros((16,), jnp.float32)
                nc[sl] = jnp.zeros((16,), jnp.int32)
                return carry

            lax.fori_loop(0, _NZ, zbody, 0)

            for c in range(_B // _CH):
                off = jb + c * _CH
                pltpu.sync_copy(idxT.at[pl.ds(off, _CH)], idxbuf)
                pltpu.sync_copy(logT.at[pl.ds(off, _CH)], logbuf)
                pltpu.sync_copy(negT.at[pl.ds(off, _CH)], negbuf)

                def abody(v, carry):
                    sl = pl.ds(v * 16, 16)
                    iv = idxbuf[sl] - base
                    m = (iv >= 0) & (iv < _HALF)
                    ivs = jnp.where(m, iv, 0)
                    plsc.addupdate_scatter(la, [ivs], logbuf[sl], mask=m)
                    plsc.addupdate_scatter(nc, [ivs], negbuf[sl], mask=m)
                    return carry

                lax.fori_loop(0, _NV, abody, 0)

            pltpu.sync_copy(la, laT_out.at[pl.ds(jm + base, _HALF)])
            pltpu.sync_copy(nc, ncT_out.at[pl.ds(jm + base, _HALF)])


def _post_body(inp_ref, laT_ref, ncT_ref, out_ref):
    sign = (1 - ((ncT_ref[...] & 1) << 1)).astype(jnp.float32)
    mult = sign * jnp.exp(laT_ref[...])
    out_ref[...] = inp_ref[...] * mult.T


_post = pl.pallas_call(
    _post_body,
    grid=(-(-_M // _POST_BR),),
    in_specs=[
        pl.BlockSpec((_POST_BR, _D), lambda i: (i, 0)),
        pl.BlockSpec((_D, _POST_BR), lambda i: (0, i)),
        pl.BlockSpec((_D, _POST_BR), lambda i: (0, i)),
    ],
    out_specs=pl.BlockSpec((_POST_BR, _D), lambda i: (i, 0)),
    out_shape=jax.ShapeDtypeStruct((_M, _D), jnp.float32),
)


def kernel(input, index, src):
    idxT, logT, negT = _pre(index, src)
    laT, ncT = _sc_scatter(idxT.reshape(-1), logT.reshape(-1), negT.reshape(-1))
    return _post(input, laT.reshape(_D, _M), ncT.reshape(_D, _M))


# speedup vs baseline: 1.3713x; 1.3713x over previous
import functools

import jax
import jax.numpy as jnp
from jax import lax
from jax.experimental import pallas as pl
from jax.experimental.pallas import tpu as pltpu
from jax.experimental.pallas import tpu_sc as plsc

_M = 100000
_B = 16384
_D = 64
_NT = 32
_HALF = _M // 2
_CH = 2048
_NV = _CH // 16
_NZ = _HALF // 16

_PRE_BR = 2048
_POST_BR = 4096


def _pre_body(idx_ref, src_ref, idxT_ref, logT_ref, negT_ref):
    s = src_ref[...]
    idxT_ref[...] = idx_ref[...].T
    logT_ref[...] = jnp.log(jnp.abs(s)).T
    negT_ref[...] = (s < 0).astype(jnp.int32).T


_pre = pl.pallas_call(
    _pre_body,
    grid=(_B // _PRE_BR,),
    in_specs=[
        pl.BlockSpec((_PRE_BR, _D), lambda i: (i, 0)),
        pl.BlockSpec((_PRE_BR, _D), lambda i: (i, 0)),
    ],
    out_specs=[
        pl.BlockSpec((_D, _PRE_BR), lambda i: (0, i)),
        pl.BlockSpec((_D, _PRE_BR), lambda i: (0, i)),
        pl.BlockSpec((_D, _PRE_BR), lambda i: (0, i)),
    ],
    out_shape=[
        jax.ShapeDtypeStruct((_D, _B), jnp.int32),
        jax.ShapeDtypeStruct((_D, _B), jnp.float32),
        jax.ShapeDtypeStruct((_D, _B), jnp.int32),
    ],
)


_mesh = plsc.VectorSubcoreMesh(core_axis_name="c", subcore_axis_name="s")


_NCH = _B // _CH
_UZ = 5
_UA = 4


@functools.partial(
    pl.kernel,
    mesh=_mesh,
    compiler_params=pltpu.CompilerParams(needs_layout_passes=False),
    out_type=[
        jax.ShapeDtypeStruct((_D * _M,), jnp.float32),
        jax.ShapeDtypeStruct((_D * _M,), jnp.int32),
    ],
    scratch_types=[
        pltpu.VMEM((_HALF,), jnp.float32),
        pltpu.VMEM((_HALF,), jnp.int32),
        pltpu.VMEM((_CH,), jnp.int32),
        pltpu.VMEM((_CH,), jnp.int32),
        pltpu.VMEM((_CH,), jnp.float32),
        pltpu.VMEM((_CH,), jnp.float32),
        pltpu.VMEM((_CH,), jnp.int32),
        pltpu.VMEM((_CH,), jnp.int32),
        pltpu.SemaphoreType.DMA,
        pltpu.SemaphoreType.DMA,
        pltpu.SemaphoreType.DMA,
    ],
)
def _sc_scatter(idxT, logT, negT, laT_out, ncT_out, la, nc,
                idxb0, idxb1, logb0, logb1, negb0, negb1, sem0, sem1, semw):
    wid = lax.axis_index("s") * 2 + lax.axis_index("c")
    bufs = ((idxb0, logb0, negb0, sem0), (idxb1, logb1, negb1, sem1))

    def start_load(jb, c):
        ib, lb, nb, sem = bufs[c % 2]
        off = jb + c * _CH
        h1 = pltpu.async_copy(idxT.at[pl.ds(off, _CH)], ib, sem)
        h2 = pltpu.async_copy(logT.at[pl.ds(off, _CH)], lb, sem)
        h3 = pltpu.async_copy(negT.at[pl.ds(off, _CH)], nb, sem)
        return (h1, h2, h3)

    first = True
    for c2 in range(2):
        j = wid + _NT * c2
        jb = j * _B
        jm = j * _M
        for p in range(2):
            base = p * _HALF
            pend = start_load(jb, 0)

            if not first:
                for h in pend_out:
                    h.wait()
            first = False

            def zbody(v, carry):
                for u in range(_UZ):
                    sl = pl.ds((v * _UZ + u) * 16, 16)
                    la[sl] = jnp.zeros((16,), jnp.float32)
                    nc[sl] = jnp.zeros((16,), jnp.int32)
                return carry

            lax.fori_loop(0, _NZ // _UZ, zbody, 0)

            for c in range(_NCH):
                cur = pend
                if c + 1 < _NCH:
                    pend = start_load(jb, c + 1)
                for h in cur:
                    h.wait()
                ib, lb, nb, _ = bufs[c % 2]

                def abody(v, carry):
                    for u in range(_UA):
                        sl = pl.ds((v * _UA + u) * 16, 16)
                        iv = ib[sl] - base
                        m = (iv >= 0) & (iv < _HALF)
                        ivs = jnp.where(m, iv, 0)
                        plsc.addupdate_scatter(la, [ivs], lb[sl], mask=m)
                        plsc.addupdate_scatter(nc, [ivs], nb[sl], mask=m)
                    return carry

                lax.fori_loop(0, _NV // _UA, abody, 0)

            pend_out = (
                pltpu.async_copy(la, laT_out.at[pl.ds(jm + base, _HALF)], semw),
                pltpu.async_copy(nc, ncT_out.at[pl.ds(jm + base, _HALF)], semw),
            )
    for h in pend_out:
        h.wait()


def _post_body(inp_ref, laT_ref, ncT_ref, out_ref):
    sign = (1 - ((ncT_ref[...] & 1) << 1)).astype(jnp.float32)
    mult = sign * jnp.exp(laT_ref[...])
    out_ref[...] = inp_ref[...] * mult.T


_post = pl.pallas_call(
    _post_body,
    grid=(-(-_M // _POST_BR),),
    in_specs=[
        pl.BlockSpec((_POST_BR, _D), lambda i: (i, 0)),
        pl.BlockSpec((_D, _POST_BR), lambda i: (0, i)),
        pl.BlockSpec((_D, _POST_BR), lambda i: (0, i)),
    ],
    out_specs=pl.BlockSpec((_POST_BR, _D), lambda i: (i, 0)),
    out_shape=jax.ShapeDtypeStruct((_M, _D), jnp.float32),
)


def kernel(input, index, src):
    idxT, logT, negT = _pre(index, src)
    laT, ncT = _sc_scatter(idxT.reshape(-1), logT.reshape(-1), negT.reshape(-1))
    return _post(input, laT.reshape(_D, _M), ncT.reshape(_D, _M))


# speedup vs baseline: 1.7940x; 1.3083x over previous
import functools

import jax
import jax.numpy as jnp
from jax import lax
from jax.experimental import pallas as pl
from jax.experimental.pallas import tpu as pltpu
from jax.experimental.pallas import tpu_sc as plsc

_M = 100000
_B = 16384
_D = 64
_NT = 32
_HALF = _M // 2
_CH = 2048
_NV = _CH // 16
_NZ = _HALF // 16

_PRE_BR = 2048
_POST_BR = 4096


def _pre_body(idx_ref, src_ref, idxT_ref, logT_ref, negT_ref):
    s = src_ref[...]
    idxT_ref[...] = idx_ref[...].T
    logT_ref[...] = jnp.log(jnp.abs(s)).T
    negT_ref[...] = (s < 0).astype(jnp.int32).T


_pre = pl.pallas_call(
    _pre_body,
    grid=(_B // _PRE_BR,),
    in_specs=[
        pl.BlockSpec((_PRE_BR, _D), lambda i: (i, 0)),
        pl.BlockSpec((_PRE_BR, _D), lambda i: (i, 0)),
    ],
    out_specs=[
        pl.BlockSpec((_D, _PRE_BR), lambda i: (0, i)),
        pl.BlockSpec((_D, _PRE_BR), lambda i: (0, i)),
        pl.BlockSpec((_D, _PRE_BR), lambda i: (0, i)),
    ],
    out_shape=[
        jax.ShapeDtypeStruct((_D, _B), jnp.int32),
        jax.ShapeDtypeStruct((_D, _B), jnp.float32),
        jax.ShapeDtypeStruct((_D, _B), jnp.int32),
    ],
)


_mesh = plsc.VectorSubcoreMesh(core_axis_name="c", subcore_axis_name="s")


_NCH = _B // _CH
_UA = 4
_TCOLS = 782
_TH = _TCOLS // 2
_HALFP = _TH * 128


@functools.partial(
    pl.kernel,
    mesh=_mesh,
    compiler_params=pltpu.CompilerParams(needs_layout_passes=False),
    out_type=[
        jax.ShapeDtypeStruct((_D // 8, _TCOLS, 8, 128), jnp.float32),
        jax.ShapeDtypeStruct((_D // 8, _TCOLS, 8, 128), jnp.int32),
    ],
    scratch_types=[
        pltpu.VMEM((_TH, 128), jnp.float32),
        pltpu.VMEM((_TH, 128), jnp.int32),
        pltpu.VMEM((_CH,), jnp.int32),
        pltpu.VMEM((_CH,), jnp.int32),
        pltpu.VMEM((_CH,), jnp.float32),
        pltpu.VMEM((_CH,), jnp.float32),
        pltpu.VMEM((_CH,), jnp.int32),
        pltpu.VMEM((_CH,), jnp.int32),
        pltpu.SemaphoreType.DMA,
        pltpu.SemaphoreType.DMA,
        pltpu.SemaphoreType.DMA,
    ],
)
def _sc_scatter(idxT, logT, negT, laT_out, ncT_out, la, nc,
                idxb0, idxb1, logb0, logb1, negb0, negb1, sem0, sem1, semw):
    wid = lax.axis_index("s") * 2 + lax.axis_index("c")
    bufs = ((idxb0, logb0, negb0, sem0), (idxb1, logb1, negb1, sem1))

    def start_load(jb, c):
        ib, lb, nb, sem = bufs[c % 2]
        off = jb + c * _CH
        h1 = pltpu.async_copy(idxT.at[pl.ds(off, _CH)], ib, sem)
        h2 = pltpu.async_copy(logT.at[pl.ds(off, _CH)], lb, sem)
        h3 = pltpu.async_copy(negT.at[pl.ds(off, _CH)], nb, sem)
        return (h1, h2, h3)

    first = True
    for c2 in range(2):
        j = wid + _NT * c2
        jb = j * _B
        jg = lax.shift_right_logical(j, 3)
        js = lax.bitwise_and(j, 7)
        for p in range(2):
            base = p * _HALFP
            pend = start_load(jb, 0)

            if not first:
                for h in pend_out:
                    h.wait()
            first = False

            def zbody(v, carry):
                for u in range(8):
                    sl = pl.ds(u * 16, 16)
                    la[v, sl] = jnp.zeros((16,), jnp.float32)
                    nc[v, sl] = jnp.zeros((16,), jnp.int32)
                return carry

            lax.fori_loop(0, _TH, zbody, 0)

            for c in range(_NCH):
                cur = pend
                if c + 1 < _NCH:
                    pend = start_load(jb, c + 1)
                for h in cur:
                    h.wait()
                ib, lb, nb, _ = bufs[c % 2]

                def abody(v, carry):
                    for u in range(_UA):
                        sl = pl.ds((v * _UA + u) * 16, 16)
                        iv = ib[sl] - base
                        m = (iv >= 0) & (iv < _HALFP)
                        ivs = jnp.where(m, iv, 0)
                        tcv = lax.shift_right_logical(ivs, 7)
                        lnv = lax.bitwise_and(ivs, 127)
                        plsc.addupdate_scatter(la, [tcv, lnv], lb[sl], mask=m)
                        plsc.addupdate_scatter(nc, [tcv, lnv], nb[sl], mask=m)
                    return carry

                lax.fori_loop(0, _NV // _UA, abody, 0)

            tc0 = p * _TH
            pend_out = (
                pltpu.async_copy(la, laT_out.at[jg, pl.ds(tc0, _TH), js, :], semw),
                pltpu.async_copy(nc, ncT_out.at[jg, pl.ds(tc0, _TH), js, :], semw),
            )
    for h in pend_out:
        h.wait()


_TB = _POST_BR // 128


def _untile(x4):
    pieces = []
    for jg in range(8):
        pieces.append(jnp.transpose(x4[jg], (1, 0, 2)).reshape(8, _POST_BR))
    return jnp.concatenate(pieces, axis=0)


def _post_body(inp_ref, laT_ref, ncT_ref, out_ref):
    la = _untile(laT_ref[...])
    nc = _untile(ncT_ref[...])
    sign = (1 - ((nc & 1) << 1)).astype(jnp.float32)
    mult = sign * jnp.exp(la)
    out_ref[...] = inp_ref[...] * mult.T


_post = pl.pallas_call(
    _post_body,
    grid=(-(-_M // _POST_BR),),
    in_specs=[
        pl.BlockSpec((_POST_BR, _D), lambda i: (i, 0)),
        pl.BlockSpec((8, _TB, 8, 128), lambda i: (0, i, 0, 0)),
        pl.BlockSpec((8, _TB, 8, 128), lambda i: (0, i, 0, 0)),
    ],
    out_specs=pl.BlockSpec((_POST_BR, _D), lambda i: (i, 0)),
    out_shape=jax.ShapeDtypeStruct((_M, _D), jnp.float32),
)


def kernel(input, index, src):
    idxT, logT, negT = _pre(index, src)
    laT, ncT = _sc_scatter(idxT.reshape(-1), logT.reshape(-1), negT.reshape(-1))
    return _post(input, laT, ncT)


# speedup vs baseline: 2.6883x; 1.4985x over previous
import functools

import jax
import jax.numpy as jnp
from jax import lax
from jax.experimental import pallas as pl
from jax.experimental.pallas import tpu as pltpu
from jax.experimental.pallas import tpu_sc as plsc

_M = 100000
_B = 16384
_D = 64
_NT = 32

_BT = _B // 128
_CHT = 16
_CH = _CHT * 128
_NCH = _BT // _CHT

_TCOLS = 782
_TH = _TCOLS // 2
_HALFP = _TH * 128

_PRE_TCH = 16
_POST_BR = 4096
_TB = _POST_BR // 128


def _tile4(x):
    w = x.shape[1]
    pieces = []
    for jg in range(8):
        pieces.append(jnp.transpose(x[jg * 8:(jg + 1) * 8].reshape(8, w // 128, 128), (1, 0, 2)))
    return jnp.stack(pieces, axis=0)


def _untile4(x4):
    t = x4.shape[1]
    pieces = []
    for jg in range(8):
        pieces.append(jnp.transpose(x4[jg], (1, 0, 2)).reshape(8, t * 128))
    return jnp.concatenate(pieces, axis=0)


def _pre_body(idxT_ref, srcT_ref, idx4_ref, log4_ref, neg4_ref):
    s = srcT_ref[...]
    idx4_ref[...] = _tile4(idxT_ref[...])
    log4_ref[...] = _tile4(jnp.log(jnp.abs(s)))
    neg4_ref[...] = _tile4((s < 0).astype(jnp.int32))


_pre = pl.pallas_call(
    _pre_body,
    grid=(_BT // _PRE_TCH,),
    in_specs=[
        pl.BlockSpec((_D, _PRE_TCH * 128), lambda i: (0, i)),
        pl.BlockSpec((_D, _PRE_TCH * 128), lambda i: (0, i)),
    ],
    out_specs=[
        pl.BlockSpec((8, _PRE_TCH, 8, 128), lambda i: (0, i, 0, 0)),
        pl.BlockSpec((8, _PRE_TCH, 8, 128), lambda i: (0, i, 0, 0)),
        pl.BlockSpec((8, _PRE_TCH, 8, 128), lambda i: (0, i, 0, 0)),
    ],
    out_shape=[
        jax.ShapeDtypeStruct((8, _BT, 8, 128), jnp.int32),
        jax.ShapeDtypeStruct((8, _BT, 8, 128), jnp.float32),
        jax.ShapeDtypeStruct((8, _BT, 8, 128), jnp.int32),
    ],
)


_mesh = plsc.VectorSubcoreMesh(core_axis_name="c", subcore_axis_name="s")


@functools.partial(
    pl.kernel,
    mesh=_mesh,
    compiler_params=pltpu.CompilerParams(needs_layout_passes=False),
    out_type=[
        jax.ShapeDtypeStruct((_D // 8, _TCOLS, 8, 128), jnp.float32),
        jax.ShapeDtypeStruct((_D // 8, _TCOLS, 8, 128), jnp.int32),
    ],
    scratch_types=[
        pltpu.VMEM((_TH, 128), jnp.float32),
        pltpu.VMEM((_TH, 128), jnp.int32),
        pltpu.VMEM((_CHT, 128), jnp.int32),
        pltpu.VMEM((_CHT, 128), jnp.int32),
        pltpu.VMEM((_CHT, 128), jnp.float32),
        pltpu.VMEM((_CHT, 128), jnp.float32),
        pltpu.VMEM((_CHT, 128), jnp.int32),
        pltpu.VMEM((_CHT, 128), jnp.int32),
        pltpu.SemaphoreType.DMA,
        pltpu.SemaphoreType.DMA,
        pltpu.SemaphoreType.DMA,
    ],
)
def _sc_scatter(idx4, log4, neg4, laT_out, ncT_out, la, nc,
                idxb0, idxb1, logb0, logb1, negb0, negb1, sem0, sem1, semw):
    wid = lax.axis_index("s") * 2 + lax.axis_index("c")
    bufs = ((idxb0, logb0, negb0, sem0), (idxb1, logb1, negb1, sem1))

    def start_load(jg, js, c):
        ib, lb, nb, sem = bufs[c % 2]
        tc = c * _CHT
        h1 = pltpu.async_copy(idx4.at[jg, pl.ds(tc, _CHT), js, :], ib, sem)
        h2 = pltpu.async_copy(log4.at[jg, pl.ds(tc, _CHT), js, :], lb, sem)
        h3 = pltpu.async_copy(neg4.at[jg, pl.ds(tc, _CHT), js, :], nb, sem)
        return (h1, h2, h3)

    first = True
    for c2 in range(2):
        j = wid + _NT * c2
        jg = lax.shift_right_logical(j, 3)
        js = lax.bitwise_and(j, 7)
        for p in range(2):
            base = p * _HALFP
            pend = start_load(jg, js, 0)

            if not first:
                for h in pend_out:
                    h.wait()
            first = False

            def zbody(v, carry):
                for u in range(8):
                    sl = pl.ds(u * 16, 16)
                    la[v, sl] = jnp.zeros((16,), jnp.float32)
                    nc[v, sl] = jnp.zeros((16,), jnp.int32)
                return carry

            lax.fori_loop(0, _TH, zbody, 0)

            for c in range(_NCH):
                cur = pend
                if c + 1 < _NCH:
                    pend = start_load(jg, js, c + 1)
                for h in cur:
                    h.wait()
                ib, lb, nb, _ = bufs[c % 2]

                def abody(t, carry):
                    for u in range(8):
                        sl = pl.ds(u * 16, 16)
                        iv = ib[t, sl] - base
                        m = (iv >= 0) & (iv < _HALFP)
                        ivs = jnp.where(m, iv, 0)
                        tcv = lax.shift_right_logical(ivs, 7)
                        lnv = lax.bitwise_and(ivs, 127)
                        plsc.addupdate_scatter(la, [tcv, lnv], lb[t, sl], mask=m)
                        plsc.addupdate_scatter(nc, [tcv, lnv], nb[t, sl], mask=m)
                    return carry

                lax.fori_loop(0, _CHT, abody, 0)

            tc0 = p * _TH
            pend_out = (
                pltpu.async_copy(la, laT_out.at[jg, pl.ds(tc0, _TH), js, :], semw),
                pltpu.async_copy(nc, ncT_out.at[jg, pl.ds(tc0, _TH), js, :], semw),
            )
    for h in pend_out:
        h.wait()


def _post_body(inpT_ref, laT_ref, ncT_ref, outT_ref):
    la = _untile4(laT_ref[...])
    nc = _untile4(ncT_ref[...])
    sign = (1 - ((nc & 1) << 1)).astype(jnp.float32)
    mult = sign * jnp.exp(la)
    outT_ref[...] = inpT_ref[...] * mult


_post = pl.pallas_call(
    _post_body,
    grid=(-(-_M // _POST_BR),),
    in_specs=[
        pl.BlockSpec((_D, _POST_BR), lambda i: (0, i)),
        pl.BlockSpec((8, _TB, 8, 128), lambda i: (0, i, 0, 0)),
        pl.BlockSpec((8, _TB, 8, 128), lambda i: (0, i, 0, 0)),
    ],
    out_specs=pl.BlockSpec((_D, _POST_BR), lambda i: (0, i)),
    out_shape=jax.ShapeDtypeStruct((_D, _M), jnp.float32),
)


def kernel(input, index, src):
    idx4, log4, neg4 = _pre(index.T, src.T)
    laT, ncT = _sc_scatter(idx4, log4, neg4)
    outT = _post(input.T, laT, ncT)
    return outT.T


# speedup vs baseline: 2.7545x; 1.0246x over previous
import functools

import jax
import jax.numpy as jnp
from jax import lax
from jax.experimental import pallas as pl
from jax.experimental.pallas import tpu as pltpu
from jax.experimental.pallas import tpu_sc as plsc

_M = 100000
_B = 16384
_D = 64
_NT = 32

_BT = _B // 128
_CHT = 16
_CH = _CHT * 128
_NCH = _BT // _CHT

_TCOLS = 782
_TH = _TCOLS // 2
_HALFP = _TH * 128

_PRE_TCH = 16
_POST_BR = 4096
_TB = _POST_BR // 128


def _tile4(x):
    w = x.shape[1]
    pieces = []
    for jg in range(8):
        pieces.append(jnp.transpose(x[jg * 8:(jg + 1) * 8].reshape(8, w // 128, 128), (1, 0, 2)))
    return jnp.stack(pieces, axis=0)


def _untile4(x4):
    g, t = x4.shape[0], x4.shape[1]
    pieces = []
    for jg in range(g):
        pieces.append(jnp.transpose(x4[jg], (1, 0, 2)).reshape(8, t * 128))
    return jnp.concatenate(pieces, axis=0)


def _pre_body(idxT_ref, srcT_ref, idx4_ref, log4_ref, neg4_ref):
    s = srcT_ref[...]
    idx4_ref[...] = _tile4(idxT_ref[...])
    log4_ref[...] = _tile4(jnp.log(jnp.abs(s)))
    neg4_ref[...] = _tile4((s < 0).astype(jnp.int32))


_pre = pl.pallas_call(
    _pre_body,
    grid=(_BT // _PRE_TCH,),
    in_specs=[
        pl.BlockSpec((_D, _PRE_TCH * 128), lambda i: (0, i)),
        pl.BlockSpec((_D, _PRE_TCH * 128), lambda i: (0, i)),
    ],
    out_specs=[
        pl.BlockSpec((8, _PRE_TCH, 8, 128), lambda i: (0, i, 0, 0)),
        pl.BlockSpec((8, _PRE_TCH, 8, 128), lambda i: (0, i, 0, 0)),
        pl.BlockSpec((8, _PRE_TCH, 8, 128), lambda i: (0, i, 0, 0)),
    ],
    out_shape=[
        jax.ShapeDtypeStruct((8, _BT, 8, 128), jnp.int32),
        jax.ShapeDtypeStruct((8, _BT, 8, 128), jnp.float32),
        jax.ShapeDtypeStruct((8, _BT, 8, 128), jnp.int32),
    ],
)


_mesh = plsc.VectorSubcoreMesh(core_axis_name="c", subcore_axis_name="s")


def _make_sc_half(jg0):
    @functools.partial(
        pl.kernel,
        mesh=_mesh,
        compiler_params=pltpu.CompilerParams(needs_layout_passes=False),
        out_type=[
            jax.ShapeDtypeStruct((4, _TCOLS, 8, 128), jnp.float32),
            jax.ShapeDtypeStruct((4, _TCOLS, 8, 128), jnp.int32),
        ],
        scratch_types=[
            pltpu.VMEM((_TH, 128), jnp.float32),
            pltpu.VMEM((_TH, 128), jnp.int32),
            pltpu.VMEM((_CHT, 128), jnp.int32),
            pltpu.VMEM((_CHT, 128), jnp.int32),
            pltpu.VMEM((_CHT, 128), jnp.float32),
            pltpu.VMEM((_CHT, 128), jnp.float32),
            pltpu.VMEM((_CHT, 128), jnp.int32),
            pltpu.VMEM((_CHT, 128), jnp.int32),
            pltpu.SemaphoreType.DMA,
            pltpu.SemaphoreType.DMA,
            pltpu.SemaphoreType.DMA,
        ],
        name=f"sc_scatter_h{jg0}",
    )
    def _sc_scatter(idx4, log4, neg4, la_out, nc_out, la, nc,
                    idxb0, idxb1, logb0, logb1, negb0, negb1, sem0, sem1, semw):
        wid = lax.axis_index("s") * 2 + lax.axis_index("c")
        bufs = ((idxb0, logb0, negb0, sem0), (idxb1, logb1, negb1, sem1))
        jgl = lax.shift_right_logical(wid, 3)
        jg = jgl + jg0
        js = lax.bitwise_and(wid, 7)

        def start_load(c):
            ib, lb, nb, sem = bufs[c % 2]
            tc = c * _CHT
            h1 = pltpu.async_copy(idx4.at[jg, pl.ds(tc, _CHT), js, :], ib, sem)
            h2 = pltpu.async_copy(log4.at[jg, pl.ds(tc, _CHT), js, :], lb, sem)
            h3 = pltpu.async_copy(neg4.at[jg, pl.ds(tc, _CHT), js, :], nb, sem)
            return (h1, h2, h3)

        first = True
        for p in range(2):
            base = p * _HALFP
            pend = start_load(0)

            if not first:
                for h in pend_out:
                    h.wait()
            first = False

            def zbody(v, carry):
                for u in range(8):
                    sl = pl.ds(u * 16, 16)
                    la[v, sl] = jnp.zeros((16,), jnp.float32)
                    nc[v, sl] = jnp.zeros((16,), jnp.int32)
                return carry

            lax.fori_loop(0, _TH, zbody, 0)

            for c in range(_NCH):
                cur = pend
                if c + 1 < _NCH:
                    pend = start_load(c + 1)
                for h in cur:
                    h.wait()
                ib, lb, nb, _ = bufs[c % 2]

                def abody(t, carry):
                    for u in range(8):
                        sl = pl.ds(u * 16, 16)
                        iv = ib[t, sl] - base
                        m = (iv >= 0) & (iv < _HALFP)
                        ivs = jnp.where(m, iv, 0)
                        tcv = lax.shift_right_logical(ivs, 7)
                        lnv = lax.bitwise_and(ivs, 127)
                        plsc.addupdate_scatter(la, [tcv, lnv], lb[t, sl], mask=m)
                        plsc.addupdate_scatter(nc, [tcv, lnv], nb[t, sl], mask=m)
                    return carry

                lax.fori_loop(0, _CHT, abody, 0)

            tc0 = p * _TH
            pend_out = (
                pltpu.async_copy(la, la_out.at[jgl, pl.ds(tc0, _TH), js, :], semw),
                pltpu.async_copy(nc, nc_out.at[jgl, pl.ds(tc0, _TH), js, :], semw),
            )
        for h in pend_out:
            h.wait()

    return _sc_scatter


_sc_half0 = _make_sc_half(0)
_sc_half1 = _make_sc_half(4)


def _post_body(inpT_ref, laT_ref, ncT_ref, _, outT_ref):
    la = _untile4(laT_ref[...])
    nc = _untile4(ncT_ref[...])
    sign = (1 - ((nc & 1) << 1)).astype(jnp.float32)
    mult = sign * jnp.exp(la)
    outT_ref[...] = inpT_ref[...] * mult


def _make_post(half):
    return pl.pallas_call(
        _post_body,
        grid=(-(-_M // _POST_BR),),
        in_specs=[
            pl.BlockSpec((_D // 2, _POST_BR), lambda i, h=half: (h, i)),
            pl.BlockSpec((4, _TB, 8, 128), lambda i: (0, i, 0, 0)),
            pl.BlockSpec((4, _TB, 8, 128), lambda i: (0, i, 0, 0)),
            pl.BlockSpec(memory_space=pl.ANY),
        ],
        out_specs=pl.BlockSpec((_D // 2, _POST_BR), lambda i, h=half: (h, i)),
        out_shape=jax.ShapeDtypeStruct((_D, _M), jnp.float32),
        input_output_aliases={3: 0},
    )


_post0 = _make_post(0)
_post1 = _make_post(1)


def kernel(input, index, src):
    idx4, log4, neg4 = _pre(index.T, src.T)
    inpT = input.T
    la0, nc0 = _sc_half0(idx4, log4, neg4)
    la1, nc1 = _sc_half1(idx4, log4, neg4)
    acc = jnp.zeros((_D, _M), jnp.float32)
    acc = _post0(inpT, la0, nc0, acc)
    outT = _post1(inpT, la1, nc1, acc)
    return outT.T


# speedup vs baseline: 2.8935x; 1.0504x over previous
import functools

import jax
import jax.numpy as jnp
from jax import lax
from jax.experimental import pallas as pl
from jax.experimental.pallas import tpu as pltpu
from jax.experimental.pallas import tpu_sc as plsc

_M = 100000
_B = 16384
_D = 64
_NT = 32

_BT = _B // 128
_CHT = 32
_CH = _CHT * 128
_NCH = _BT // _CHT

_TCOLS = 782
_TH = _TCOLS // 2
_HALFP = _TH * 128

_PRE_TCH = 16
_POST_BR = 4096
_TB = _POST_BR // 128


def _tile4(x):
    w = x.shape[1]
    pieces = []
    for jg in range(8):
        pieces.append(jnp.transpose(x[jg * 8:(jg + 1) * 8].reshape(8, w // 128, 128), (1, 0, 2)))
    return jnp.stack(pieces, axis=0)


def _untile4(x4):
    g, t = x4.shape[0], x4.shape[1]
    pieces = []
    for jg in range(g):
        pieces.append(jnp.transpose(x4[jg], (1, 0, 2)).reshape(8, t * 128))
    return jnp.concatenate(pieces, axis=0)


def _pre_body(idxT_ref, srcT_ref, idx4_ref, log4_ref, neg4_ref):
    s = srcT_ref[...]
    idx4_ref[...] = _tile4(idxT_ref[...])
    log4_ref[...] = _tile4(jnp.log(jnp.abs(s)))
    neg4_ref[...] = _tile4((s < 0).astype(jnp.int32))


_pre = pl.pallas_call(
    _pre_body,
    grid=(_BT // _PRE_TCH,),
    in_specs=[
        pl.BlockSpec((_D, _PRE_TCH * 128), lambda i: (0, i)),
        pl.BlockSpec((_D, _PRE_TCH * 128), lambda i: (0, i)),
    ],
    out_specs=[
        pl.BlockSpec((8, _PRE_TCH, 8, 128), lambda i: (0, i, 0, 0)),
        pl.BlockSpec((8, _PRE_TCH, 8, 128), lambda i: (0, i, 0, 0)),
        pl.BlockSpec((8, _PRE_TCH, 8, 128), lambda i: (0, i, 0, 0)),
    ],
    out_shape=[
        jax.ShapeDtypeStruct((8, _BT, 8, 128), jnp.int32),
        jax.ShapeDtypeStruct((8, _BT, 8, 128), jnp.float32),
        jax.ShapeDtypeStruct((8, _BT, 8, 128), jnp.int32),
    ],
)


_mesh = plsc.VectorSubcoreMesh(core_axis_name="c", subcore_axis_name="s")


def _make_sc_half(jg0):
    @functools.partial(
        pl.kernel,
        mesh=_mesh,
        compiler_params=pltpu.CompilerParams(needs_layout_passes=False),
        out_type=[
            jax.ShapeDtypeStruct((4, _TCOLS, 8, 128), jnp.float32),
            jax.ShapeDtypeStruct((4, _TCOLS, 8, 128), jnp.int32),
        ],
        scratch_types=[
            pltpu.VMEM((_TH, 128), jnp.float32),
            pltpu.VMEM((_TH, 128), jnp.int32),
            pltpu.VMEM((_CHT, 128), jnp.int32),
            pltpu.VMEM((_CHT, 128), jnp.int32),
            pltpu.VMEM((_CHT, 128), jnp.float32),
            pltpu.VMEM((_CHT, 128), jnp.float32),
            pltpu.VMEM((_CHT, 128), jnp.int32),
            pltpu.VMEM((_CHT, 128), jnp.int32),
            pltpu.SemaphoreType.DMA,
            pltpu.SemaphoreType.DMA,
            pltpu.SemaphoreType.DMA,
        ],
        name=f"sc_scatter_h{jg0}",
    )
    def _sc_scatter(idx4, log4, neg4, la_out, nc_out, la, nc,
                    idxb0, idxb1, logb0, logb1, negb0, negb1, sem0, sem1, semw):
        wid = lax.axis_index("s") * 2 + lax.axis_index("c")
        bufs = ((idxb0, logb0, negb0, sem0), (idxb1, logb1, negb1, sem1))
        jgl = lax.shift_right_logical(wid, 3)
        jg = jgl + jg0
        js = lax.bitwise_and(wid, 7)

        def start_load(c):
            ib, lb, nb, sem = bufs[c % 2]
            tc = c * _CHT
            h1 = pltpu.async_copy(idx4.at[jg, pl.ds(tc, _CHT), js, :], ib, sem)
            h2 = pltpu.async_copy(log4.at[jg, pl.ds(tc, _CHT), js, :], lb, sem)
            h3 = pltpu.async_copy(neg4.at[jg, pl.ds(tc, _CHT), js, :], nb, sem)
            return (h1, h2, h3)

        first = True
        for p in range(2):
            base = p * _HALFP
            pend = start_load(0)

            if not first:
                pend_out[0].wait()

            def zla(v, carry):
                for u in range(8):
                    la[v, pl.ds(u * 16, 16)] = jnp.zeros((16,), jnp.float32)
                return carry

            lax.fori_loop(0, _TH, zla, 0)

            if not first:
                pend_out[1].wait()
            first = False

            def znc(v, carry):
                for u in range(8):
                    nc[v, pl.ds(u * 16, 16)] = jnp.zeros((16,), jnp.int32)
                return carry

            lax.fori_loop(0, _TH, znc, 0)

            for c in range(_NCH):
                cur = pend
                if c + 1 < _NCH:
                    pend = start_load(c + 1)
                for h in cur:
                    h.wait()
                ib, lb, nb, _ = bufs[c % 2]

                def abody(t, carry):
                    for u in range(8):
                        sl = pl.ds(u * 16, 16)
                        iv = ib[t, sl] - base
                        m = (iv >= 0) & (iv < _HALFP)
                        ivs = jnp.where(m, iv, 0)
                        tcv = lax.shift_right_logical(ivs, 7)
                        lnv = lax.bitwise_and(ivs, 127)
                        plsc.addupdate_scatter(la, [tcv, lnv], lb[t, sl], mask=m)
                        plsc.addupdate_scatter(nc, [tcv, lnv], nb[t, sl], mask=m)
                    return carry

                lax.fori_loop(0, _CHT, abody, 0)

            tc0 = p * _TH
            pend_out = (
                pltpu.async_copy(la, la_out.at[jgl, pl.ds(tc0, _TH), js, :], semw),
                pltpu.async_copy(nc, nc_out.at[jgl, pl.ds(tc0, _TH), js, :], semw),
            )
        for h in pend_out:
            h.wait()

    return _sc_scatter


_sc_half0 = _make_sc_half(0)
_sc_half1 = _make_sc_half(4)


def _post_body0(inpT_ref, laT_ref, ncT_ref, outT_ref):
    la = _untile4(laT_ref[...])
    nc = _untile4(ncT_ref[...])
    sign = (1 - ((nc & 1) << 1)).astype(jnp.float32)
    mult = sign * jnp.exp(la)
    outT_ref[...] = inpT_ref[...] * mult


def _post_body1(inpT_ref, laT_ref, ncT_ref, _, outT_ref):
    _post_body0(inpT_ref, laT_ref, ncT_ref, outT_ref)


_post0 = pl.pallas_call(
    _post_body0,
    grid=(-(-_M // _POST_BR),),
    in_specs=[
        pl.BlockSpec((_D // 2, _POST_BR), lambda i: (0, i)),
        pl.BlockSpec((4, _TB, 8, 128), lambda i: (0, i, 0, 0)),
        pl.BlockSpec((4, _TB, 8, 128), lambda i: (0, i, 0, 0)),
    ],
    out_specs=pl.BlockSpec((_D // 2, _POST_BR), lambda i: (0, i)),
    out_shape=jax.ShapeDtypeStruct((_D, _M), jnp.float32),
)

_post1 = pl.pallas_call(
    _post_body1,
    grid=(-(-_M // _POST_BR),),
    in_specs=[
        pl.BlockSpec((_D // 2, _POST_BR), lambda i: (1, i)),
        pl.BlockSpec((4, _TB, 8, 128), lambda i: (0, i, 0, 0)),
        pl.BlockSpec((4, _TB, 8, 128), lambda i: (0, i, 0, 0)),
        pl.BlockSpec(memory_space=pl.ANY),
    ],
    out_specs=pl.BlockSpec((_D // 2, _POST_BR), lambda i: (1, i)),
    out_shape=jax.ShapeDtypeStruct((_D, _M), jnp.float32),
    input_output_aliases={3: 0},
)


def kernel(input, index, src):
    idx4, log4, neg4 = _pre(index.T, src.T)
    inpT = input.T
    la0, nc0 = _sc_half0(idx4, log4, neg4)
    la1, nc1 = _sc_half1(idx4, log4, neg4)
    acc = _post0(inpT, la0, nc0)
    outT = _post1(inpT, la1, nc1, acc)
    return outT.T


# speedup vs baseline: 2.9647x; 1.0246x over previous
import functools

import jax
import jax.numpy as jnp
from jax import lax
from jax.experimental import pallas as pl
from jax.experimental.pallas import tpu as pltpu
from jax.experimental.pallas import tpu_sc as plsc

_M = 100000
_B = 16384
_D = 64
_NT = 32

_BT = _B // 128
_CHT = 32
_CH = _CHT * 128
_NCH = _BT // _CHT

_TCOLS = 782
_TH = _TCOLS // 2
_HALFP = _TH * 128

_PRE_TCH = 16
_POST_BR = 4096
_TB = _POST_BR // 128


def _tile4(x):
    g, w = x.shape[0] // 8, x.shape[1]
    pieces = []
    for jg in range(g):
        pieces.append(jnp.transpose(x[jg * 8:(jg + 1) * 8].reshape(8, w // 128, 128), (1, 0, 2)))
    return jnp.stack(pieces, axis=0)


def _untile4(x4):
    g, t = x4.shape[0], x4.shape[1]
    pieces = []
    for jg in range(g):
        pieces.append(jnp.transpose(x4[jg], (1, 0, 2)).reshape(8, t * 128))
    return jnp.concatenate(pieces, axis=0)


def _pre_body(idxT_ref, srcT_ref, idx4_ref, log4_ref, neg4_ref):
    s = srcT_ref[...]
    idx4_ref[...] = _tile4(idxT_ref[...])
    log4_ref[...] = _tile4(jnp.log(jnp.abs(s)))
    neg4_ref[...] = _tile4((s < 0).astype(jnp.int32))


def _make_pre(half):
    return pl.pallas_call(
        _pre_body,
        grid=(_BT // _PRE_TCH,),
        in_specs=[
            pl.BlockSpec((_D // 2, _PRE_TCH * 128), lambda i, h=half: (h, i)),
            pl.BlockSpec((_D // 2, _PRE_TCH * 128), lambda i, h=half: (h, i)),
        ],
        out_specs=[
            pl.BlockSpec((4, _PRE_TCH, 8, 128), lambda i: (0, i, 0, 0)),
            pl.BlockSpec((4, _PRE_TCH, 8, 128), lambda i: (0, i, 0, 0)),
            pl.BlockSpec((4, _PRE_TCH, 8, 128), lambda i: (0, i, 0, 0)),
        ],
        out_shape=[
            jax.ShapeDtypeStruct((4, _BT, 8, 128), jnp.int32),
            jax.ShapeDtypeStruct((4, _BT, 8, 128), jnp.float32),
            jax.ShapeDtypeStruct((4, _BT, 8, 128), jnp.int32),
        ],
    )


_pre0 = _make_pre(0)
_pre1 = _make_pre(1)


_mesh = plsc.VectorSubcoreMesh(core_axis_name="c", subcore_axis_name="s")


def _make_sc_half(jg0):
    @functools.partial(
        pl.kernel,
        mesh=_mesh,
        compiler_params=pltpu.CompilerParams(needs_layout_passes=False),
        out_type=[
            jax.ShapeDtypeStruct((4, _TCOLS, 8, 128), jnp.float32),
            jax.ShapeDtypeStruct((4, _TCOLS, 8, 128), jnp.int32),
        ],
        scratch_types=[
            pltpu.VMEM((_TH, 128), jnp.float32),
            pltpu.VMEM((_TH, 128), jnp.int32),
            pltpu.VMEM((_CHT, 128), jnp.int32),
            pltpu.VMEM((_CHT, 128), jnp.int32),
            pltpu.VMEM((_CHT, 128), jnp.float32),
            pltpu.VMEM((_CHT, 128), jnp.float32),
            pltpu.VMEM((_CHT, 128), jnp.int32),
            pltpu.VMEM((_CHT, 128), jnp.int32),
            pltpu.SemaphoreType.DMA,
            pltpu.SemaphoreType.DMA,
            pltpu.SemaphoreType.DMA,
        ],
        name=f"sc_scatter_h{jg0}",
    )
    def _sc_scatter(idx4, log4, neg4, la_out, nc_out, la, nc,
                    idxb0, idxb1, logb0, logb1, negb0, negb1, sem0, sem1, semw):
        wid = lax.axis_index("s") * 2 + lax.axis_index("c")
        bufs = ((idxb0, logb0, negb0, sem0), (idxb1, logb1, negb1, sem1))
        jgl = lax.shift_right_logical(wid, 3)
        js = lax.bitwise_and(wid, 7)

        def start_load(c):
            ib, lb, nb, sem = bufs[c % 2]
            tc = c * _CHT
            h1 = pltpu.async_copy(idx4.at[jgl, pl.ds(tc, _CHT), js, :], ib, sem)
            h2 = pltpu.async_copy(log4.at[jgl, pl.ds(tc, _CHT), js, :], lb, sem)
            h3 = pltpu.async_copy(neg4.at[jgl, pl.ds(tc, _CHT), js, :], nb, sem)
            return (h1, h2, h3)

        first = True
        for p in range(2):
            base = p * _HALFP
            pend = start_load(0)

            if not first:
                pend_out[0].wait()

            def zla(v, carry):
                for u in range(8):
                    la[v, pl.ds(u * 16, 16)] = jnp.zeros((16,), jnp.float32)
                return carry

            lax.fori_loop(0, _TH, zla, 0)

            if not first:
                pend_out[1].wait()
            first = False

            def znc(v, carry):
                for u in range(8):
                    nc[v, pl.ds(u * 16, 16)] = jnp.zeros((16,), jnp.int32)
                return carry

            lax.fori_loop(0, _TH, znc, 0)

            for c in range(_NCH):
                cur = pend
                if c + 1 < _NCH:
                    pend = start_load(c + 1)
                for h in cur:
                    h.wait()
                ib, lb, nb, _ = bufs[c % 2]

                def abody(t, carry):
                    for u in range(8):
                        sl = pl.ds(u * 16, 16)
                        iv = ib[t, sl] - base
                        m = (iv >= 0) & (iv < _HALFP)
                        ivs = jnp.where(m, iv, 0)
                        tcv = lax.shift_right_logical(ivs, 7)
                        lnv = lax.bitwise_and(ivs, 127)
                        plsc.addupdate_scatter(la, [tcv, lnv], lb[t, sl], mask=m)
                        plsc.addupdate_scatter(nc, [tcv, lnv], nb[t, sl], mask=m)
                    return carry

                lax.fori_loop(0, _CHT, abody, 0)

            tc0 = p * _TH
            pend_out = (
                pltpu.async_copy(la, la_out.at[jgl, pl.ds(tc0, _TH), js, :], semw),
                pltpu.async_copy(nc, nc_out.at[jgl, pl.ds(tc0, _TH), js, :], semw),
            )
        for h in pend_out:
            h.wait()

    return _sc_scatter


_sc_half0 = _make_sc_half(0)
_sc_half1 = _make_sc_half(4)


def _post_body0(inpT_ref, laT_ref, ncT_ref, outT_ref):
    la = _untile4(laT_ref[...])
    nc = _untile4(ncT_ref[...])
    sign = (1 - ((nc & 1) << 1)).astype(jnp.float32)
    mult = sign * jnp.exp(la)
    outT_ref[...] = inpT_ref[...] * mult


def _post_body1(inpT_ref, laT_ref, ncT_ref, _, outT_ref):
    _post_body0(inpT_ref, laT_ref, ncT_ref, outT_ref)


_post0 = pl.pallas_call(
    _post_body0,
    grid=(-(-_M // _POST_BR),),
    in_specs=[
        pl.BlockSpec((_D // 2, _POST_BR), lambda i: (0, i)),
        pl.BlockSpec((4, _TB, 8, 128), lambda i: (0, i, 0, 0)),
        pl.BlockSpec((4, _TB, 8, 128), lambda i: (0, i, 0, 0)),
    ],
    out_specs=pl.BlockSpec((_D // 2, _POST_BR), lambda i: (0, i)),
    out_shape=jax.ShapeDtypeStruct((_D, _M), jnp.float32),
)

_post1 = pl.pallas_call(
    _post_body1,
    grid=(-(-_M // _POST_BR),),
    in_specs=[
        pl.BlockSpec((_D // 2, _POST_BR), lambda i: (1, i)),
        pl.BlockSpec((4, _TB, 8, 128), lambda i: (0, i, 0, 0)),
        pl.BlockSpec((4, _TB, 8, 128), lambda i: (0, i, 0, 0)),
        pl.BlockSpec(memory_space=pl.ANY),
    ],
    out_specs=pl.BlockSpec((_D // 2, _POST_BR), lambda i: (1, i)),
    out_shape=jax.ShapeDtypeStruct((_D, _M), jnp.float32),
    input_output_aliases={3: 0},
)


def kernel(input, index, src):
    idxT, srcT = index.T, src.T
    idx4a, log4a, neg4a = _pre0(idxT, srcT)
    idx4b, log4b, neg4b = _pre1(idxT, srcT)
    inpT = input.T
    la0, nc0 = _sc_half0(idx4a, log4a, neg4a)
    la1, nc1 = _sc_half1(idx4b, log4b, neg4b)
    acc = _post0(inpT, la0, nc0)
    outT = _post1(inpT, la1, nc1, acc)
    return outT.T


# speedup vs baseline: 3.1362x; 1.0578x over previous
import functools

import jax
import jax.numpy as jnp
from jax import lax
from jax.experimental import pallas as pl
from jax.experimental.pallas import tpu as pltpu
from jax.experimental.pallas import tpu_sc as plsc

_M = 100000
_B = 16384
_D = 64
_NT = 32

_BT = _B // 128
_CHT = 32
_CH = _CHT * 128
_NCH = _BT // _CHT

_TCOLS = 782
_TH = _TCOLS // 2
_HALFP = _TH * 128

_PRE_TCH = 16
_POST_BR = 8192
_TB = _POST_BR // 128


def _tile4(x):
    g, w = x.shape[0] // 8, x.shape[1]
    pieces = []
    for jg in range(g):
        pieces.append(jnp.transpose(x[jg * 8:(jg + 1) * 8].reshape(8, w // 128, 128), (1, 0, 2)))
    return jnp.stack(pieces, axis=0)


def _untile4(x4):
    g, t = x4.shape[0], x4.shape[1]
    pieces = []
    for jg in range(g):
        pieces.append(jnp.transpose(x4[jg], (1, 0, 2)).reshape(8, t * 128))
    return jnp.concatenate(pieces, axis=0)


def _pre_body(idxT_ref, srcT_ref, idx4_ref, log4_ref, neg4_ref):
    s = srcT_ref[...]
    idx4_ref[...] = _tile4(idxT_ref[...])
    log4_ref[...] = _tile4(jnp.log(jnp.abs(s)))
    neg4_ref[...] = _tile4((s < 0).astype(jnp.int32))


def _make_pre(half):
    return pl.pallas_call(
        _pre_body,
        grid=(_BT // _PRE_TCH,),
        in_specs=[
            pl.BlockSpec((_D // 2, _PRE_TCH * 128), lambda i, h=half: (h, i)),
            pl.BlockSpec((_D // 2, _PRE_TCH * 128), lambda i, h=half: (h, i)),
        ],
        out_specs=[
            pl.BlockSpec((4, _PRE_TCH, 8, 128), lambda i: (0, i, 0, 0)),
            pl.BlockSpec((4, _PRE_TCH, 8, 128), lambda i: (0, i, 0, 0)),
            pl.BlockSpec((4, _PRE_TCH, 8, 128), lambda i: (0, i, 0, 0)),
        ],
        out_shape=[
            jax.ShapeDtypeStruct((4, _BT, 8, 128), jnp.int32),
            jax.ShapeDtypeStruct((4, _BT, 8, 128), jnp.float32),
            jax.ShapeDtypeStruct((4, _BT, 8, 128), jnp.int32),
        ],
    )


_pre0 = _make_pre(0)
_pre1 = _make_pre(1)


_mesh = plsc.VectorSubcoreMesh(core_axis_name="c", subcore_axis_name="s")


def _make_sc_half(jg0):
    @functools.partial(
        pl.kernel,
        mesh=_mesh,
        compiler_params=pltpu.CompilerParams(needs_layout_passes=False),
        out_type=[
            jax.ShapeDtypeStruct((4, _TCOLS, 8, 128), jnp.float32),
            jax.ShapeDtypeStruct((4, _TCOLS, 8, 128), jnp.int32),
        ],
        scratch_types=[
            pltpu.VMEM((_TH, 128), jnp.float32),
            pltpu.VMEM((_TH, 128), jnp.int32),
            pltpu.VMEM((_CHT, 128), jnp.int32),
            pltpu.VMEM((_CHT, 128), jnp.int32),
            pltpu.VMEM((_CHT, 128), jnp.float32),
            pltpu.VMEM((_CHT, 128), jnp.float32),
            pltpu.VMEM((_CHT, 128), jnp.int32),
            pltpu.VMEM((_CHT, 128), jnp.int32),
            pltpu.SemaphoreType.DMA,
            pltpu.SemaphoreType.DMA,
            pltpu.SemaphoreType.DMA,
        ],
        name=f"sc_scatter_h{jg0}",
    )
    def _sc_scatter(idx4, log4, neg4, la_out, nc_out, la, nc,
                    idxb0, idxb1, logb0, logb1, negb0, negb1, sem0, sem1, semw):
        wid = lax.axis_index("s") * 2 + lax.axis_index("c")
        bufs = ((idxb0, logb0, negb0, sem0), (idxb1, logb1, negb1, sem1))
        jgl = lax.shift_right_logical(wid, 3)
        js = lax.bitwise_and(wid, 7)

        def start_load(c):
            ib, lb, nb, sem = bufs[c % 2]
            tc = c * _CHT
            h1 = pltpu.async_copy(idx4.at[jgl, pl.ds(tc, _CHT), js, :], ib, sem)
            h2 = pltpu.async_copy(log4.at[jgl, pl.ds(tc, _CHT), js, :], lb, sem)
            h3 = pltpu.async_copy(neg4.at[jgl, pl.ds(tc, _CHT), js, :], nb, sem)
            return (h1, h2, h3)

        first = True
        for p in range(2):
            base = p * _HALFP
            pend = start_load(0)

            if not first:
                pend_out[0].wait()

            def zla(v, carry):
                for u in range(8):
                    la[v, pl.ds(u * 16, 16)] = jnp.zeros((16,), jnp.float32)
                return carry

            lax.fori_loop(0, _TH, zla, 0)

            if not first:
                pend_out[1].wait()
            first = False

            def znc(v, carry):
                for u in range(8):
                    nc[v, pl.ds(u * 16, 16)] = jnp.zeros((16,), jnp.int32)
                return carry

            lax.fori_loop(0, _TH, znc, 0)

            for c in range(_NCH):
                cur = pend
                if c + 1 < _NCH:
                    pend = start_load(c + 1)
                for h in cur:
                    h.wait()
                ib, lb, nb, _ = bufs[c % 2]

                def abody(t, carry):
                    for u in range(8):
                        sl = pl.ds(u * 16, 16)
                        iv = ib[t, sl] - base
                        m = (iv >= 0) & (iv < _HALFP)
                        ivs = jnp.where(m, iv, 0)
                        tcv = lax.shift_right_logical(ivs, 7)
                        lnv = lax.bitwise_and(ivs, 127)
                        plsc.addupdate_scatter(la, [tcv, lnv], lb[t, sl], mask=m)
                        plsc.addupdate_scatter(nc, [tcv, lnv], nb[t, sl], mask=m)
                    return carry

                lax.fori_loop(0, _CHT, abody, 0)

            tc0 = p * _TH
            pend_out = (
                pltpu.async_copy(la, la_out.at[jgl, pl.ds(tc0, _TH), js, :], semw),
                pltpu.async_copy(nc, nc_out.at[jgl, pl.ds(tc0, _TH), js, :], semw),
            )
        for h in pend_out:
            h.wait()

    return _sc_scatter


_sc_half0 = _make_sc_half(0)
_sc_half1 = _make_sc_half(4)


def _post_body0(inpT_ref, laT_ref, ncT_ref, outT_ref):
    la = _untile4(laT_ref[...])
    nc = _untile4(ncT_ref[...])
    sign = (1 - ((nc & 1) << 1)).astype(jnp.float32)
    mult = sign * jnp.exp(la)
    outT_ref[...] = inpT_ref[...] * mult


def _post_body1(inpT_ref, laT_ref, ncT_ref, _, outT_ref):
    _post_body0(inpT_ref, laT_ref, ncT_ref, outT_ref)


_post0 = pl.pallas_call(
    _post_body0,
    grid=(-(-_M // _POST_BR),),
    in_specs=[
        pl.BlockSpec((_D // 2, _POST_BR), lambda i: (0, i)),
        pl.BlockSpec((4, _TB, 8, 128), lambda i: (0, i, 0, 0)),
        pl.BlockSpec((4, _TB, 8, 128), lambda i: (0, i, 0, 0)),
    ],
    out_specs=pl.BlockSpec((_D // 2, _POST_BR), lambda i: (0, i)),
    out_shape=jax.ShapeDtypeStruct((_D, _M), jnp.float32),
)

_post1 = pl.pallas_call(
    _post_body1,
    grid=(-(-_M // _POST_BR),),
    in_specs=[
        pl.BlockSpec((_D // 2, _POST_BR), lambda i: (1, i)),
        pl.BlockSpec((4, _TB, 8, 128), lambda i: (0, i, 0, 0)),
        pl.BlockSpec((4, _TB, 8, 128), lambda i: (0, i, 0, 0)),
        pl.BlockSpec(memory_space=pl.ANY),
    ],
    out_specs=pl.BlockSpec((_D // 2, _POST_BR), lambda i: (1, i)),
    out_shape=jax.ShapeDtypeStruct((_D, _M), jnp.float32),
    input_output_aliases={3: 0},
)


def kernel(input, index, src):
    idxT, srcT = index.T, src.T
    idx4a, log4a, neg4a = _pre0(idxT, srcT)
    idx4b, log4b, neg4b = _pre1(idxT, srcT)
    inpT = input.T
    la0, nc0 = _sc_half0(idx4a, log4a, neg4a)
    la1, nc1 = _sc_half1(idx4b, log4b, neg4b)
    acc = _post0(inpT, la0, nc0)
    outT = _post1(inpT, la1, nc1, acc)
    return outT.T
